# Initial kernel scaffold; baseline (speedup 1.0000x reference)
import jax
import jax.numpy as jnp
from jax.experimental import pallas as pl


def kernel(visit_tensor, table):
    raise NotImplementedError("write your pallas kernel here")


# speedup vs baseline: 10.6428x; 10.6428x over previous
import functools

import jax
import jax.numpy as jnp
from jax import lax
from jax.experimental import pallas as pl
from jax.experimental.pallas import tpu as pltpu
from jax.experimental.pallas import tpu_sc as plsc

BSZ = 1024
NVISITS = 50
VISIT = 26
ED = 64
SEGS = BSZ * NVISITS
NC, NS = 2, 16
NW = NC * NS
SEG_PER_W = SEGS // NW
CHUNK = 32
ROWS = CHUNK * VISIT
FIRES = 8
FIRE_ROWS = ROWS // FIRES
NCHUNK = SEG_PER_W // CHUNK
LANES = 16
NVREG = ED // LANES


def _embed_grid():
    mesh = plsc.VectorSubcoreMesh(core_axis_name="c", subcore_axis_name="s")

    @functools.partial(
        pl.kernel,
        mesh=mesh,
        compiler_params=pltpu.CompilerParams(use_tc_tiling_on_sc=False),
        out_type=jax.ShapeDtypeStruct((SEGS, ED), jnp.float32),
        scratch_types=[
            pltpu.VMEM((FIRES, FIRE_ROWS), jnp.int32),
            pltpu.VMEM((ROWS, ED), jnp.float32),
            pltpu.VMEM((CHUNK, ED), jnp.float32),
            pltpu.SemaphoreType.DMA,
        ],
    )
    def body(idx_hbm, table_hbm, out_hbm, idx_v, rows_v, out_v, sem):
        wid = lax.axis_index("s") * NC + lax.axis_index("c")
        base_seg = wid * SEG_PER_W

        def chunk_body(i, carry):
            seg0 = base_seg + i * CHUNK
            irow0 = pl.multiple_of(seg0 * VISIT // FIRE_ROWS, 8)
            pltpu.sync_copy(idx_hbm.at[pl.ds(irow0, FIRES)], idx_v)

            for j in range(FIRES):
                pltpu.async_copy(
                    table_hbm.at[idx_v.at[j]],
                    rows_v.at[pl.ds(j * FIRE_ROWS, FIRE_ROWS)],
                    sem,
                )
            for j in range(FIRES):
                pltpu.make_async_copy(
                    table_hbm.at[idx_v.at[j]],
                    rows_v.at[pl.ds(j * FIRE_ROWS, FIRE_ROWS)],
                    sem,
                ).wait()

            def seg_body(s, c2):
                r0 = s * VISIT
                for cc in range(NVREG):
                    sl = pl.ds(cc * LANES, LANES)
                    acc = rows_v[r0, sl]
                    for r in range(1, VISIT):
                        acc = acc + rows_v[r0 + r, sl]
                    out_v[s, sl] = acc
                return c2

            lax.fori_loop(0, CHUNK, seg_body, 0)
            pltpu.sync_copy(out_v, out_hbm.at[pl.ds(seg0, CHUNK)])
            return carry

        lax.fori_loop(0, NCHUNK, chunk_body, 0)

    return body


_EMBED = _embed_grid()


def kernel(visit_tensor, table):
    flat_idx = visit_tensor.reshape(SEGS * VISIT // FIRE_ROWS, FIRE_ROWS)
    out = _EMBED(flat_idx, table)
    return out.reshape(BSZ, NVISITS, ED)


# speedup vs baseline: 17.2362x; 1.6195x over previous
import functools

import jax
import jax.numpy as jnp
from jax import lax
from jax.experimental import pallas as pl
from jax.experimental.pallas import tpu as pltpu
from jax.experimental.pallas import tpu_sc as plsc

BSZ = 1024
NVISITS = 50
VISIT = 26
ED = 64
SEGS = BSZ * NVISITS
NC, NS = 2, 16
NW = NC * NS
SEG_PER_W = SEGS // NW
CHUNK = 64
ROWS = CHUNK * VISIT
FIRES = 16
FIRE_ROWS = ROWS // FIRES
NCHUNK = SEG_PER_W // CHUNK
BLANES = 32
NVREG = ED // BLANES


def _embed_grid():
    mesh = plsc.VectorSubcoreMesh(core_axis_name="c", subcore_axis_name="s")

    @functools.partial(
        pl.kernel,
        mesh=mesh,
        compiler_params=pltpu.CompilerParams(use_tc_tiling_on_sc=False),
        out_type=jax.ShapeDtypeStruct((SEGS, ED), jnp.bfloat16),
        scratch_types=[
            pltpu.VMEM((2, FIRES, FIRE_ROWS), jnp.int32),
            pltpu.VMEM((2, ROWS, ED), jnp.bfloat16),
            pltpu.VMEM((2, CHUNK, ED), jnp.bfloat16),
            pltpu.SemaphoreType.DMA,
            pltpu.SemaphoreType.DMA,
            pltpu.SemaphoreType.DMA,
            pltpu.SemaphoreType.DMA,
        ],
    )
    def body(idx_hbm, table_hbm, out_hbm, idx_v, rows_v, out_v, sem_a, sem_b,
             sem_oa, sem_ob):
        wid = lax.axis_index("s") * NC + lax.axis_index("c")
        base_seg = wid * SEG_PER_W

        def fire(i, b, sem):
            seg0 = base_seg + i * CHUNK
            irow0 = pl.multiple_of(seg0 * VISIT // FIRE_ROWS, 8)
            pltpu.sync_copy(idx_hbm.at[pl.ds(irow0, FIRES)], idx_v.at[b])
            for j in range(FIRES):
                pltpu.async_copy(
                    table_hbm.at[idx_v.at[b, j]],
                    rows_v.at[b, pl.ds(j * FIRE_ROWS, FIRE_ROWS)],
                    sem,
                )

        def drain(b, sem):
            for j in range(FIRES):
                pltpu.make_async_copy(
                    table_hbm.at[idx_v.at[b, j]],
                    rows_v.at[b, pl.ds(j * FIRE_ROWS, FIRE_ROWS)],
                    sem,
                ).wait()

        def out_desc(b, sem):
            return pltpu.make_async_copy(
                out_v.at[b], out_hbm.at[pl.ds(base_seg, CHUNK)], sem)

        def compute_store(i, b, sem):
            seg0 = base_seg + i * CHUNK

            def seg_body(s, c2):
                r0 = s * VISIT
                for cc in range(NVREG):
                    sl = pl.ds(cc * BLANES, BLANES)
                    vals = [rows_v[b, r0 + r, sl] for r in range(VISIT)]
                    while len(vals) > 1:
                        nxt = [vals[k] + vals[k + 1]
                               for k in range(0, len(vals) - 1, 2)]
                        if len(vals) % 2:
                            nxt[-1] = nxt[-1] + vals[-1]
                        vals = nxt
                    out_v[b, s, sl] = vals[0]
                return c2

            lax.fori_loop(0, CHUNK, seg_body, 0)
            pltpu.async_copy(out_v.at[b], out_hbm.at[pl.ds(seg0, CHUNK)], sem)

        fire(0, 0, sem_a)

        def pair_body(i2, carry):
            i0 = i2 * 2
            fire(i0 + 1, 1, sem_b)
            drain(0, sem_a)

            @pl.when(i2 > 0)
            def _():
                out_desc(0, sem_oa).wait()

            compute_store(i0, 0, sem_oa)

            @pl.when(i0 + 2 < NCHUNK)
            def _():
                fire(i0 + 2, 0, sem_a)

            drain(1, sem_b)

            @pl.when(i2 > 0)
            def _():
                out_desc(1, sem_ob).wait()

            compute_store(i0 + 1, 1, sem_ob)
            return carry

        lax.fori_loop(0, NCHUNK // 2, pair_body, 0)
        drain(0, sem_a)
        out_desc(0, sem_oa).wait()
        compute_store(NCHUNK - 1, 0, sem_oa)
        out_desc(0, sem_oa).wait()
        out_desc(1, sem_ob).wait()

    return body


_EMBED = _embed_grid()


def kernel(visit_tensor, table):
    flat_idx = visit_tensor.reshape(SEGS * VISIT // FIRE_ROWS, FIRE_ROWS)
    out = _EMBED(flat_idx, table.astype(jnp.bfloat16))
    return out.astype(jnp.float32).reshape(BSZ, NVISITS, ED)


# speedup vs baseline: 17.9376x; 1.0407x over previous
import functools

import jax
import jax.numpy as jnp
import numpy as np
from jax import lax
from jax.experimental import pallas as pl
from jax.experimental.pallas import tpu as pltpu
from jax.experimental.pallas import tpu_sc as plsc

BSZ = 1024
NVISITS = 50
VISIT = 26
ED = 64
SEGS = BSZ * NVISITS
NC, NS = 2, 16
NW = NC * NS
SEG_PER_W = SEGS // NW
CHUNK = 64
ROWS = CHUNK * VISIT
FIRES = 16
FIRE_ROWS = ROWS // FIRES
NCHUNK = SEG_PER_W // CHUNK
LANES = 16
BLANES = 32
NVREG = ED // BLANES

_PERM = np.empty((ED,), dtype=np.int32)
for _h in range(NVREG):
    for _i in range(LANES):
        _PERM[_h * BLANES + 2 * _i] = _h * BLANES + _i
        _PERM[_h * BLANES + 2 * _i + 1] = _h * BLANES + LANES + _i


def _embed_grid():
    mesh = plsc.VectorSubcoreMesh(core_axis_name="c", subcore_axis_name="s")

    @functools.partial(
        pl.kernel,
        mesh=mesh,
        compiler_params=pltpu.CompilerParams(
            use_tc_tiling_on_sc=False, needs_layout_passes=False),
        out_type=jax.ShapeDtypeStruct((SEGS, ED), jnp.float32),
        scratch_types=[
            pltpu.VMEM((2, ROWS), jnp.int32),
            pltpu.VMEM((2, ROWS, ED), jnp.bfloat16),
            pltpu.VMEM((2, CHUNK, ED), jnp.float32),
            pltpu.SemaphoreType.DMA,
            pltpu.SemaphoreType.DMA,
            pltpu.SemaphoreType.DMA,
            pltpu.SemaphoreType.DMA,
        ],
    )
    def body(idx_hbm, table_hbm, out_hbm, idx_v, rows_v, out_v, sem_a, sem_b,
             sem_oa, sem_ob):
        wid = lax.axis_index("s") * NC + lax.axis_index("c")
        base_seg = wid * SEG_PER_W

        def fire(i, b, sem):
            seg0 = base_seg + i * CHUNK
            e0 = pl.multiple_of(seg0 * VISIT, 8)
            pltpu.sync_copy(idx_hbm.at[pl.ds(e0, ROWS)], idx_v.at[b])
            for j in range(FIRES):
                pltpu.async_copy(
                    table_hbm.at[idx_v.at[b, pl.ds(j * FIRE_ROWS, FIRE_ROWS)]],
                    rows_v.at[b, pl.ds(j * FIRE_ROWS, FIRE_ROWS)],
                    sem,
                )

        def drain(b, sem):
            for j in range(FIRES):
                pltpu.make_async_copy(
                    table_hbm.at[idx_v.at[b, pl.ds(j * FIRE_ROWS, FIRE_ROWS)]],
                    rows_v.at[b, pl.ds(j * FIRE_ROWS, FIRE_ROWS)],
                    sem,
                ).wait()

        def out_desc(b, sem):
            return pltpu.make_async_copy(
                out_v.at[b], out_hbm.at[pl.ds(base_seg, CHUNK)], sem)

        def compute_store(i, b, sem):
            seg0 = base_seg + i * CHUNK

            def seg_body(s, c2):
                r0 = s * VISIT
                for cc in range(NVREG):
                    sl = pl.ds(cc * BLANES, BLANES)
                    vals = [rows_v[b, r0 + r, sl] for r in range(VISIT)]
                    while len(vals) > 1:
                        nxt = [vals[k] + vals[k + 1]
                               for k in range(0, len(vals) - 1, 2)]
                        if len(vals) % 2:
                            nxt[-1] = nxt[-1] + vals[-1]
                        vals = nxt
                    lo, hi = plsc.unpack(
                        vals[0], format=plsc.PackFormat.INTERLEAVED)
                    out_v[b, s, pl.ds(cc * BLANES, LANES)] = lo
                    out_v[b, s, pl.ds(cc * BLANES + LANES, LANES)] = hi
                return c2

            lax.fori_loop(0, CHUNK, seg_body, 0)
            pltpu.async_copy(out_v.at[b], out_hbm.at[pl.ds(seg0, CHUNK)], sem)

        fire(0, 0, sem_a)

        def pair_body(i2, carry):
            i0 = i2 * 2
            fire(i0 + 1, 1, sem_b)
            drain(0, sem_a)

            @pl.when(i2 > 0)
            def _():
                out_desc(0, sem_oa).wait()

            compute_store(i0, 0, sem_oa)

            @pl.when(i0 + 2 < NCHUNK)
            def _():
                fire(i0 + 2, 0, sem_a)

            drain(1, sem_b)

            @pl.when(i2 > 0)
            def _():
                out_desc(1, sem_ob).wait()

            compute_store(i0 + 1, 1, sem_ob)
            return carry

        lax.fori_loop(0, NCHUNK // 2, pair_body, 0)
        drain(0, sem_a)
        out_desc(0, sem_oa).wait()
        compute_store(NCHUNK - 1, 0, sem_oa)
        out_desc(0, sem_oa).wait()
        out_desc(1, sem_ob).wait()

    return body


_EMBED = _embed_grid()


def kernel(visit_tensor, table):
    flat_idx = visit_tensor.reshape(SEGS * VISIT)
    tablep = table.astype(jnp.bfloat16)[:, _PERM]
    out = _EMBED(flat_idx, tablep)
    return out.reshape(BSZ, NVISITS, ED)


# speedup vs baseline: 18.3918x; 1.0253x over previous
import functools

import jax
import jax.numpy as jnp
from jax import lax
from jax.experimental import pallas as pl
from jax.experimental.pallas import tpu as pltpu
from jax.experimental.pallas import tpu_sc as plsc

BSZ = 1024
NVISITS = 50
VISIT = 26
ED = 64
NC, NS = 2, 16
NW = NC * NS
BATCH_PER_W = BSZ // NW
NPAIR = BATCH_PER_W // 2
FIRES = 2
FIRE_VISITS = NVISITS // FIRES
LANES = 16
BLANES = 32
NVREG = ED // BLANES


def _embed_grid():
    mesh = plsc.VectorSubcoreMesh(core_axis_name="c", subcore_axis_name="s")

    @functools.partial(
        pl.kernel,
        mesh=mesh,
        compiler_params=pltpu.CompilerParams(
            use_tc_tiling_on_sc=False, needs_layout_passes=False),
        out_type=jax.ShapeDtypeStruct((BSZ, NVISITS, ED), jnp.float32),
        scratch_types=[
            pltpu.VMEM((2, 2, NVISITS, VISIT), jnp.int32),
            pltpu.VMEM((2, NVISITS, VISIT, ED), jnp.bfloat16),
            pltpu.VMEM((2, NVISITS, ED), jnp.float32),
            pltpu.SemaphoreType.DMA,
            pltpu.SemaphoreType.DMA,
            pltpu.SemaphoreType.DMA,
            pltpu.SemaphoreType.DMA,
        ],
    )
    def body(idx_hbm, table_hbm, out_hbm, idx_v, rows_v, out_v, sem_a, sem_b,
             sem_oa, sem_ob):
        wid = lax.axis_index("s") * NC + lax.axis_index("c")
        base_b = wid * BATCH_PER_W
        even = 2 * lax.broadcasted_iota(jnp.int32, (LANES,), 0)
        odd = even + 1

        def stage(k, kp):
            pltpu.sync_copy(idx_hbm.at[pl.ds(base_b + 2 * k, 2)],
                            idx_v.at[kp])

        def fire(kp, bb, g, sem):
            for v in range(NVISITS):
                pltpu.async_copy(
                    table_hbm.at[idx_v.at[kp, bb, v]],
                    rows_v.at[g, v],
                    sem,
                )

        def drain(kp, bb, g, sem):
            for v in range(NVISITS):
                pltpu.make_async_copy(
                    table_hbm.at[idx_v.at[kp, bb, v]],
                    rows_v.at[g, v],
                    sem,
                ).wait()

        def out_desc(g, sem):
            return pltpu.make_async_copy(out_v.at[g], out_hbm.at[base_b], sem)

        def compute_store(b, g, sem):
            def seg_body(s, c2):
                for cc in range(NVREG):
                    sl = pl.ds(cc * BLANES, BLANES)
                    vals = [rows_v[g, s, r, sl] for r in range(VISIT)]
                    while len(vals) > 1:
                        nxt = [vals[k] + vals[k + 1]
                               for k in range(0, len(vals) - 1, 2)]
                        if len(vals) % 2:
                            nxt[-1] = nxt[-1] + vals[-1]
                        vals = nxt
                    lo, hi = plsc.unpack(
                        vals[0], format=plsc.PackFormat.INTERLEAVED)
                    orow = out_v.at[g, s, pl.ds(cc * BLANES, BLANES)]
                    plsc.store_scatter(orow, [even], lo)
                    plsc.store_scatter(orow, [odd], hi)
                return c2

            lax.fori_loop(0, NVISITS, seg_body, 0)
            pltpu.async_copy(out_v.at[g], out_hbm.at[b], sem)

        stage(0, 0)
        fire(0, 0, 0, sem_a)

        def pair_body(k, carry):
            kp = lax.rem(k, 2)
            b0 = base_b + 2 * k
            fire(kp, 1, 1, sem_b)
            drain(kp, 0, 0, sem_a)

            @pl.when(k > 0)
            def _():
                out_desc(0, sem_oa).wait()

            compute_store(b0, 0, sem_oa)

            @pl.when(k + 1 < NPAIR)
            def _():
                stage(k + 1, 1 - kp)
                fire(1 - kp, 0, 0, sem_a)

            drain(kp, 1, 1, sem_b)

            @pl.when(k > 0)
            def _():
                out_desc(1, sem_ob).wait()

            compute_store(b0 + 1, 1, sem_ob)
            return carry

        lax.fori_loop(0, NPAIR, pair_body, 0)
        out_desc(0, sem_oa).wait()
        out_desc(1, sem_ob).wait()

    return body


_EMBED = _embed_grid()


def kernel(visit_tensor, table):
    return _EMBED(visit_tensor, table.astype(jnp.bfloat16))


# speedup vs baseline: 22.3361x; 1.2145x over previous
import functools

import jax
import jax.numpy as jnp
from jax import lax
from jax.experimental import pallas as pl
from jax.experimental.pallas import tpu as pltpu
from jax.experimental.pallas import tpu_sc as plsc

BSZ = 1024
NVISITS = 50
VISIT = 26
ED = 64
ROWS = NVISITS * VISIT
NC, NS = 2, 16
NW = NC * NS
BATCH_PER_W = BSZ // NW
NPAIR = BATCH_PER_W // 2
LANES = 16
BLANES = 32
NVREG = ED // BLANES
CHUNKS = [(j * 104, 104) for j in range(12)] + [(1248, 52)]


def _embed_grid():
    mesh = plsc.VectorSubcoreMesh(core_axis_name="c", subcore_axis_name="s")

    @functools.partial(
        pl.kernel,
        mesh=mesh,
        compiler_params=pltpu.CompilerParams(
            use_tc_tiling_on_sc=False, needs_layout_passes=False),
        out_type=jax.ShapeDtypeStruct((BSZ, NVISITS, ED), jnp.float32),
        scratch_types=[
            pltpu.VMEM((2, 2, ROWS), jnp.int32),
            pltpu.VMEM((2, ROWS, ED), jnp.bfloat16),
            pltpu.VMEM((2, NVISITS, ED), jnp.float32),
            pltpu.SemaphoreType.DMA,
            pltpu.SemaphoreType.DMA,
            pltpu.SemaphoreType.DMA,
            pltpu.SemaphoreType.DMA,
            pltpu.SemaphoreType.DMA,
        ],
    )
    def body(idx_hbm, table_hbm, out_hbm, idx_v, rows_v, out_v, sem_a, sem_b,
             sem_oa, sem_ob, sem_i):
        wid = lax.axis_index("s") * NC + lax.axis_index("c")
        base_b = wid * BATCH_PER_W
        even = 2 * lax.broadcasted_iota(jnp.int32, (LANES,), 0)
        odd = even + 1

        def stage_desc(k, kp):
            return pltpu.make_async_copy(
                idx_hbm.at[pl.ds(base_b + 2 * k, 2)], idx_v.at[kp], sem_i)

        def fire(kp, bb, g, sem):
            for off, ln in CHUNKS:
                pltpu.async_copy(
                    table_hbm.at[idx_v.at[kp, bb, pl.ds(off, ln)]],
                    rows_v.at[g, pl.ds(off, ln)],
                    sem,
                )

        def drain(kp, bb, g, sem):
            for off, ln in CHUNKS:
                pltpu.make_async_copy(
                    table_hbm.at[idx_v.at[kp, bb, pl.ds(off, ln)]],
                    rows_v.at[g, pl.ds(off, ln)],
                    sem,
                ).wait()

        def out_desc(g, sem):
            return pltpu.make_async_copy(out_v.at[g], out_hbm.at[base_b], sem)

        def compute_store(b, g, sem):
            def seg_body(s, c2):
                r0 = s * VISIT
                for cc in range(NVREG):
                    sl = pl.ds(cc * BLANES, BLANES)
                    vals = [rows_v[g, r0 + r, sl] for r in range(VISIT)]
                    while len(vals) > 1:
                        nxt = [vals[k] + vals[k + 1]
                               for k in range(0, len(vals) - 1, 2)]
                        if len(vals) % 2:
                            nxt[-1] = nxt[-1] + vals[-1]
                        vals = nxt
                    lo, hi = plsc.unpack(
                        vals[0], format=plsc.PackFormat.INTERLEAVED)
                    orow = out_v.at[g, s, pl.ds(cc * BLANES, BLANES)]
                    plsc.store_scatter(orow, [even], lo)
                    plsc.store_scatter(orow, [odd], hi)
                return c2

            lax.fori_loop(0, NVISITS, seg_body, 0)
            pltpu.async_copy(out_v.at[g], out_hbm.at[b], sem)

        d0 = stage_desc(0, 0)
        d0.start()
        d0.wait()
        fire(0, 0, 0, sem_a)

        def pair_body(k, carry):
            kp = lax.rem(k, 2)
            b0 = base_b + 2 * k
            fire(kp, 1, 1, sem_b)

            @pl.when(k + 1 < NPAIR)
            def _():
                stage_desc(k + 1, 1 - kp).start()

            drain(kp, 0, 0, sem_a)

            @pl.when(k > 0)
            def _():
                out_desc(0, sem_oa).wait()

            compute_store(b0, 0, sem_oa)

            @pl.when(k + 1 < NPAIR)
            def _():
                stage_desc(k + 1, 1 - kp).wait()
                fire(1 - kp, 0, 0, sem_a)

            drain(kp, 1, 1, sem_b)

            @pl.when(k > 0)
            def _():
                out_desc(1, sem_ob).wait()

            compute_store(b0 + 1, 1, sem_ob)
            return carry

        lax.fori_loop(0, NPAIR, pair_body, 0)
        out_desc(0, sem_oa).wait()
        out_desc(1, sem_ob).wait()

    return body


_EMBED = _embed_grid()


def kernel(visit_tensor, table):
    return _EMBED(visit_tensor.reshape(BSZ, ROWS), table.astype(jnp.bfloat16))
